# baseline (speedup 1.0000x reference)
import jax
import jax.numpy as jnp
import numpy as np
from jax.experimental import pallas as pl
from jax.experimental.pallas import tpu as pltpu

_B, _N, _K, _C, _H, _NL, _FFN = 4, 512, 20, 64, 8, 4, 128
_DH = _C // _H
_BIG = 1e9
_HI = jax.lax.Precision.HIGHEST
_BNI = 1.0 / np.sqrt(1.0 + 1e-5)


_PREC = jax.lax.Precision.DEFAULT


def _dot(a, b):
    return jax.lax.dot_general(a, b, (((1,), (0,)), ((), ())),
                               precision=_PREC,
                               preferred_element_type=jnp.float32)


def _lrelu(h):
    return jnp.where(h >= 0, h, 0.2 * h)


def _ln(t, g, b):
    m = jnp.mean(t, axis=-1, keepdims=True)
    v = jnp.mean((t - m) ** 2, axis=-1, keepdims=True)
    return (t - m) / jnp.sqrt(v + 1e-5) * g + b


def _fwd_kernel(pd_ref, xf_ref, ed_ref,
                c1w_ref, c1b_ref, bn1g_ref, bn1b_ref,
                gw_ref, gb_ref, bng_ref, bnb_ref,
                wq_ref, wk_ref, wv_ref, wo_ref,
                bq_ref, bk_ref, bv_ref, bo_ref,
                l1g_ref, l1b_ref, f1w_ref, f1b_ref,
                f2w_ref, f2b_ref, l2g_ref, l2b_ref,
                q1w_ref, q6g_ref, q6b_ref,
                q2w_ref, q2b_ref, q7g_ref, q7b_ref,
                q3w_ref, q3b_ref, out_ref):
    pd = pd_ref[0]
    iota_c = jax.lax.broadcasted_iota(jnp.int32, (_N, _N), 1)
    iota_r = jax.lax.broadcasted_iota(jnp.int32, (_N, _N), 0)
    a = jnp.zeros((_N, _N), jnp.float32)
    for k in range(_K):
        am = jnp.argmax(pd, axis=-1, keepdims=True)
        sel = iota_c == am
        a = jnp.maximum(a, sel.astype(jnp.float32))
        pd = jnp.where(sel, -jnp.inf, pd)
    at = jnp.transpose(a)
    eye = (iota_c == iota_r).astype(jnp.float32)

    deg_row = jnp.sum(a, axis=0, keepdims=True) + 1.0
    deg_col = jnp.sum(at, axis=1, keepdims=True) + 1.0
    ahat = (jax.lax.rsqrt(deg_col) * (at + eye)) * jax.lax.rsqrt(deg_row)

    asym = jnp.maximum(a, at)
    dist = jnp.where(asym > 0, jnp.float32(1.0), jnp.float32(_BIG))
    dist = jnp.where(eye > 0, jnp.float32(0.0), dist)
    reach = jnp.minimum(asym + eye, 1.0)

    def bfs_cond(st):
        t, _, _, changed = st
        return jnp.logical_and(changed, t < _N)

    asym16 = asym.astype(jnp.bfloat16)

    def bfs_body(st):
        t, d, r, _ = st
        cnt = jax.lax.dot_general(
            r.astype(jnp.bfloat16), asym16, (((1,), (0,)), ((), ())),
            preferred_element_type=jnp.float32)
        rn = jnp.maximum(r, (cnt > 0.0).astype(jnp.float32))
        new = rn > r
        d = jnp.where(new, t.astype(jnp.float32), d)
        return t + 1, d, rn, jnp.sum(rn - r) > 0.0

    _, dist, _, _ = jax.lax.while_loop(
        bfs_cond, bfs_body,
        (jnp.int32(2), dist, reach, jnp.bool_(True)))
    sidx = jnp.where(dist > 255.0, 255, dist.astype(jnp.int32))

    md = jnp.max(jnp.where(sidx == 255, jnp.int32(-1), sidx))
    bias0 = tuple(
        jnp.full((_N, _N), ed_ref[255, hh], jnp.float32) for hh in range(_H))

    def bias_body(t, bs):
        mask = sidx == t
        return tuple(
            jnp.where(mask, ed_ref[t, hh], bs[hh]) for hh in range(_H))

    biases = jax.lax.fori_loop(0, md + 1, bias_body, bias0)

    A = ahat
    xg = xf_ref[0]
    h = _dot(A, _dot(xg, c1w_ref[...])) + c1b_ref[...]
    h = _lrelu(h * _BNI * bn1g_ref[...] + bn1b_ref[...])

    isc = np.float32(1.0 / np.sqrt(_DH))
    for l in range(_NL):
        h = _dot(A, _dot(h, gw_ref[l])) + gb_ref[l]
        h = _lrelu(h * _BNI * bng_ref[l] + bnb_ref[l])
        x0 = h
        q = (_dot(x0, wq_ref[l]) + bq_ref[l]) * isc
        kk = _dot(x0, wk_ref[l]) + bk_ref[l]
        v = _dot(x0, wv_ref[l]) + bv_ref[l]
        outs = []
        for hh in range(_H):
            sl = slice(hh * _DH, (hh + 1) * _DH)
            sc = jax.lax.dot_general(
                q[:, sl], kk[:, sl], (((1,), (1,)), ((), ())),
                precision=jax.lax.Precision.DEFAULT,
                preferred_element_type=jnp.float32)
            sc = sc + biases[hh]
            m = jnp.max(sc, axis=-1, keepdims=True)
            e = jnp.exp(sc - m)
            attn = e * (1.0 / jnp.sum(e, axis=-1, keepdims=True))
            outs.append(jax.lax.dot_general(
                attn, v[:, sl], (((1,), (0,)), ((), ())),
                precision=jax.lax.Precision.DEFAULT,
                preferred_element_type=jnp.float32))
        o = jnp.concatenate(outs, axis=1)
        o = _dot(o, wo_ref[l]) + bo_ref[l]
        y = _ln(x0 + o, l1g_ref[l], l1b_ref[l])
        f = _dot(jnp.maximum(_dot(y, f1w_ref[l]) + f1b_ref[l], 0.0),
                 f2w_ref[l]) + f2b_ref[l]
        y = _ln(y + f, l2g_ref[l], l2b_ref[l])
        h = h + y

    xsum = jnp.sum(h, axis=0, keepdims=True)
    z = jnp.concatenate([xsum / _N, xsum], axis=1)
    z = _lrelu(_dot(z, q1w_ref[...]) * _BNI * q6g_ref[...] + q6b_ref[...])
    z = _lrelu((_dot(z, q2w_ref[...]) + q2b_ref[...]) * _BNI * q7g_ref[...]
               + q7b_ref[...])
    out_ref[0] = _dot(z, q3w_ref[...]) + q3b_ref[...]


def _full(shape):
    n = len(shape)
    return pl.BlockSpec(shape, lambda b: (0,) * n)


def _perb(shape):
    n = len(shape)
    return pl.BlockSpec((1,) + shape, lambda b: (b,) + (0,) * n)


def kernel(x, params):
    p = params
    inner = -2.0 * jnp.einsum('bcn,bcm->bnm', x, x)
    xx = jnp.sum(x ** 2, axis=1, keepdims=True)
    pd = -xx - inner - jnp.transpose(xx, (0, 2, 1))
    xf = jnp.transpose(x, (0, 2, 1))

    def st(fmt, reshape=None):
        a = jnp.stack([p[fmt % l] for l in range(_NL)])
        return a.reshape(reshape) if reshape else a

    args = [
        pd, xf, p['edge_dis'],
        p['conv1_W'], p['conv1_b'].reshape(1, _C),
        p['bn1_g'].reshape(1, _C), p['bn1_b'].reshape(1, _C),
        st('l%d_gcn_W'), st('l%d_gcn_b', (_NL, 1, _C)),
        st('l%d_bn_g', (_NL, 1, _C)), st('l%d_bn_b', (_NL, 1, _C)),
        st('l%d_Wq'), st('l%d_Wk'), st('l%d_Wv'), st('l%d_Wo'),
        st('l%d_bq', (_NL, 1, _C)), st('l%d_bk', (_NL, 1, _C)),
        st('l%d_bv', (_NL, 1, _C)), st('l%d_bo', (_NL, 1, _C)),
        st('l%d_ln1_g', (_NL, 1, _C)), st('l%d_ln1_b', (_NL, 1, _C)),
        st('l%d_fc1_W'), st('l%d_fc1_b', (_NL, 1, _FFN)),
        st('l%d_fc2_W'), st('l%d_fc2_b', (_NL, 1, _C)),
        st('l%d_ln2_g', (_NL, 1, _C)), st('l%d_ln2_b', (_NL, 1, _C)),
        p['lin1_W'],
        p['bn6_g'].reshape(1, 2 * _C), p['bn6_b'].reshape(1, 2 * _C),
        p['lin2_W'], p['lin2_b'].reshape(1, 2 * _C),
        p['bn7_g'].reshape(1, 2 * _C), p['bn7_b'].reshape(1, 2 * _C),
        p['lin3_W'], p['lin3_b'].reshape(1, 40),
    ]

    in_specs = [
        _perb((_N, _N)), _perb((_N, 3)),
        pl.BlockSpec(memory_space=pltpu.SMEM),
    ] + [_full(a.shape) for a in args[3:]]

    out = pl.pallas_call(
        _fwd_kernel,
        grid=(_B,),
        in_specs=in_specs,
        out_specs=_perb((1, 40)),
        out_shape=jax.ShapeDtypeStruct((_B, 1, 40), jnp.float32),
        compiler_params=pltpu.CompilerParams(
            dimension_semantics=("parallel",)),
    )(*args)
    return out.reshape(_B, 40)


# speedup vs baseline: 1.0808x; 1.0808x over previous
import jax
import jax.numpy as jnp
import numpy as np
from jax.experimental import pallas as pl
from jax.experimental.pallas import tpu as pltpu

_B, _N, _K, _C, _H, _NL, _FFN = 4, 512, 20, 64, 8, 4, 128
_DH = _C // _H
_BIG = 1e9
_HI = jax.lax.Precision.HIGHEST
_BNI = 1.0 / np.sqrt(1.0 + 1e-5)


_PREC = jax.lax.Precision.DEFAULT


def _dot(a, b):
    return jax.lax.dot_general(a, b, (((1,), (0,)), ((), ())),
                               precision=_PREC,
                               preferred_element_type=jnp.float32)


def _lrelu(h):
    return jnp.where(h >= 0, h, 0.2 * h)


def _ln(t, g, b):
    m = jnp.mean(t, axis=-1, keepdims=True)
    v = jnp.mean((t - m) ** 2, axis=-1, keepdims=True)
    return (t - m) / jnp.sqrt(v + 1e-5) * g + b


def _fwd_kernel(pd_ref, xf_ref, ed_ref,
                c1w_ref, c1b_ref, bn1g_ref, bn1b_ref,
                gw_ref, gb_ref, bng_ref, bnb_ref,
                wq_ref, wk_ref, wv_ref, wo_ref,
                bq_ref, bk_ref, bv_ref, bo_ref,
                l1g_ref, l1b_ref, f1w_ref, f1b_ref,
                f2w_ref, f2b_ref, l2g_ref, l2b_ref,
                q1w_ref, q6g_ref, q6b_ref,
                q2w_ref, q2b_ref, q7g_ref, q7b_ref,
                q3w_ref, q3b_ref, out_ref):
    pd = pd_ref[0]
    iota_c = jax.lax.broadcasted_iota(jnp.int32, (_N, _N), 1)
    iota_r = jax.lax.broadcasted_iota(jnp.int32, (_N, _N), 0)
    a = jnp.zeros((_N, _N), jnp.float32)
    for k in range(_K):
        am = jnp.argmax(pd, axis=-1, keepdims=True)
        sel = iota_c == am
        a = jnp.maximum(a, sel.astype(jnp.float32))
        pd = jnp.where(sel, -jnp.inf, pd)
    at = jnp.transpose(a)
    eye = (iota_c == iota_r).astype(jnp.float32)

    deg_row = jnp.sum(a, axis=0, keepdims=True) + 1.0
    deg_col = jnp.sum(at, axis=1, keepdims=True) + 1.0
    ahat = (jax.lax.rsqrt(deg_col) * (at + eye)) * jax.lax.rsqrt(deg_row)

    asym = jnp.maximum(a, at)
    asym16 = asym.astype(jnp.bfloat16)
    bias0 = tuple(
        jnp.where(eye > 0, ed_ref[0, hh],
                  jnp.where(asym > 0, ed_ref[1, hh], ed_ref[255, hh]))
        for hh in range(_H))
    reach = jnp.minimum(asym + eye, 1.0).astype(jnp.bfloat16)

    def bfs_cond(st):
        t, _, _, changed = st
        return jnp.logical_and(changed, t < _N)

    def bfs_body(st):
        t, r, bs, _ = st
        cnt = jax.lax.dot_general(
            r, asym16, (((1,), (0,)), ((), ())),
            preferred_element_type=jnp.float32)
        rn = jnp.maximum(r, (cnt > 0.0).astype(jnp.bfloat16))
        new = rn > r
        tc = jnp.minimum(t, 255)
        bs = tuple(
            jnp.where(new, ed_ref[tc, hh], b) for hh, b in enumerate(bs))
        return t + 1, rn, bs, jnp.sum((rn - r).astype(jnp.float32)) > 0.0

    _, _, biases, _ = jax.lax.while_loop(
        bfs_cond, bfs_body,
        (jnp.int32(2), reach, bias0, jnp.bool_(True)))

    A = ahat
    xg = xf_ref[0]
    h = _dot(A, _dot(xg, c1w_ref[...])) + c1b_ref[...]
    h = _lrelu(h * _BNI * bn1g_ref[...] + bn1b_ref[...])

    isc = np.float32(1.0 / np.sqrt(_DH))
    for l in range(_NL):
        h = _dot(A, _dot(h, gw_ref[l])) + gb_ref[l]
        h = _lrelu(h * _BNI * bng_ref[l] + bnb_ref[l])
        x0 = h
        q = (_dot(x0, wq_ref[l]) + bq_ref[l]) * isc
        kk = _dot(x0, wk_ref[l]) + bk_ref[l]
        v = _dot(x0, wv_ref[l]) + bv_ref[l]
        outs = []
        for hh in range(_H):
            sl = slice(hh * _DH, (hh + 1) * _DH)
            sc = jax.lax.dot_general(
                q[:, sl], kk[:, sl], (((1,), (1,)), ((), ())),
                precision=jax.lax.Precision.DEFAULT,
                preferred_element_type=jnp.float32)
            sc = sc + biases[hh]
            m = jnp.max(sc, axis=-1, keepdims=True)
            e = jnp.exp(sc - m)
            attn = e / jnp.sum(e, axis=-1, keepdims=True)
            outs.append(jax.lax.dot_general(
                attn, v[:, sl], (((1,), (0,)), ((), ())),
                precision=jax.lax.Precision.DEFAULT,
                preferred_element_type=jnp.float32))
        o = jnp.concatenate(outs, axis=1)
        o = _dot(o, wo_ref[l]) + bo_ref[l]
        y = _ln(x0 + o, l1g_ref[l], l1b_ref[l])
        f = _dot(jnp.maximum(_dot(y, f1w_ref[l]) + f1b_ref[l], 0.0),
                 f2w_ref[l]) + f2b_ref[l]
        y = _ln(y + f, l2g_ref[l], l2b_ref[l])
        h = h + y

    xsum = jnp.sum(h, axis=0, keepdims=True)
    z = jnp.concatenate([xsum / _N, xsum], axis=1)
    z = _lrelu(_dot(z, q1w_ref[...]) * _BNI * q6g_ref[...] + q6b_ref[...])
    z = _lrelu((_dot(z, q2w_ref[...]) + q2b_ref[...]) * _BNI * q7g_ref[...]
               + q7b_ref[...])
    out_ref[0] = _dot(z, q3w_ref[...]) + q3b_ref[...]


def _full(shape):
    n = len(shape)
    return pl.BlockSpec(shape, lambda b: (0,) * n)


def _perb(shape):
    n = len(shape)
    return pl.BlockSpec((1,) + shape, lambda b: (b,) + (0,) * n)


def kernel(x, params):
    p = params
    inner = -2.0 * jnp.einsum('bcn,bcm->bnm', x, x)
    xx = jnp.sum(x ** 2, axis=1, keepdims=True)
    pd = -xx - inner - jnp.transpose(xx, (0, 2, 1))
    xf = jnp.transpose(x, (0, 2, 1))

    def st(fmt, reshape=None):
        a = jnp.stack([p[fmt % l] for l in range(_NL)])
        return a.reshape(reshape) if reshape else a

    args = [
        pd, xf, p['edge_dis'],
        p['conv1_W'], p['conv1_b'].reshape(1, _C),
        p['bn1_g'].reshape(1, _C), p['bn1_b'].reshape(1, _C),
        st('l%d_gcn_W'), st('l%d_gcn_b', (_NL, 1, _C)),
        st('l%d_bn_g', (_NL, 1, _C)), st('l%d_bn_b', (_NL, 1, _C)),
        st('l%d_Wq'), st('l%d_Wk'), st('l%d_Wv'), st('l%d_Wo'),
        st('l%d_bq', (_NL, 1, _C)), st('l%d_bk', (_NL, 1, _C)),
        st('l%d_bv', (_NL, 1, _C)), st('l%d_bo', (_NL, 1, _C)),
        st('l%d_ln1_g', (_NL, 1, _C)), st('l%d_ln1_b', (_NL, 1, _C)),
        st('l%d_fc1_W'), st('l%d_fc1_b', (_NL, 1, _FFN)),
        st('l%d_fc2_W'), st('l%d_fc2_b', (_NL, 1, _C)),
        st('l%d_ln2_g', (_NL, 1, _C)), st('l%d_ln2_b', (_NL, 1, _C)),
        p['lin1_W'],
        p['bn6_g'].reshape(1, 2 * _C), p['bn6_b'].reshape(1, 2 * _C),
        p['lin2_W'], p['lin2_b'].reshape(1, 2 * _C),
        p['bn7_g'].reshape(1, 2 * _C), p['bn7_b'].reshape(1, 2 * _C),
        p['lin3_W'], p['lin3_b'].reshape(1, 40),
    ]

    in_specs = [
        _perb((_N, _N)), _perb((_N, 3)),
        pl.BlockSpec(memory_space=pltpu.SMEM),
    ] + [_full(a.shape) for a in args[3:]]

    out = pl.pallas_call(
        _fwd_kernel,
        grid=(_B,),
        in_specs=in_specs,
        out_specs=_perb((1, 40)),
        out_shape=jax.ShapeDtypeStruct((_B, 1, 40), jnp.float32),
        compiler_params=pltpu.CompilerParams(
            dimension_semantics=("parallel",)),
    )(*args)
    return out.reshape(_B, 40)


# speedup vs baseline: 1.0826x; 1.0017x over previous
import jax
import jax.numpy as jnp
import numpy as np
from jax.experimental import pallas as pl
from jax.experimental.pallas import tpu as pltpu

_B, _N, _K, _C, _H, _NL, _FFN = 4, 512, 20, 64, 8, 4, 128
_DH = _C // _H
_BNI = 1.0 / np.sqrt(1.0 + 1e-5)
_PREC = jax.lax.Precision.DEFAULT


def _dot(a, b):
    return jax.lax.dot_general(a, b, (((1,), (0,)), ((), ())),
                               precision=_PREC,
                               preferred_element_type=jnp.float32)


def _lrelu(h):
    return jnp.where(h >= 0, h, 0.2 * h)


def _ln(t, g, b):
    m = jnp.mean(t, axis=-1, keepdims=True)
    v = jnp.mean((t - m) ** 2, axis=-1, keepdims=True)
    return (t - m) / jnp.sqrt(v + 1e-5) * g + b


def _fwd_kernel(pd_ref, xf_ref, ed_ref,
                c1w_ref, c1b_ref, bn1g_ref, bn1b_ref,
                gw_ref, gb_ref, bng_ref, bnb_ref,
                wq_ref, wk_ref, wv_ref, wo_ref,
                bq_ref, bk_ref, bv_ref, bo_ref,
                l1g_ref, l1b_ref, f1w_ref, f1b_ref,
                f2w_ref, f2b_ref, l2g_ref, l2b_ref,
                q1w_ref, q6g_ref, q6b_ref,
                q2w_ref, q2b_ref, q7g_ref, q7b_ref,
                q3w_ref, q3b_ref, out_ref):
    pd = pd_ref[0]
    iota_c = jax.lax.broadcasted_iota(jnp.int32, (_N, _N), 1)
    iota_r = jax.lax.broadcasted_iota(jnp.int32, (_N, _N), 0)
    a = jnp.zeros((_N, _N), jnp.float32)
    for k in range(_K):
        am = jnp.argmax(pd, axis=-1, keepdims=True)
        sel = iota_c == am
        a = jnp.maximum(a, sel.astype(jnp.float32))
        pd = jnp.where(sel, -jnp.inf, pd)
    at = jnp.transpose(a)
    eye = (iota_c == iota_r).astype(jnp.float32)

    deg_row = jnp.sum(a, axis=0, keepdims=True) + 1.0
    deg_col = jnp.sum(at, axis=1, keepdims=True) + 1.0
    ahat = (jax.lax.rsqrt(deg_col) * (at + eye)) * jax.lax.rsqrt(deg_row)

    asym = jnp.maximum(a, at)
    asym16 = asym.astype(jnp.bfloat16)
    bias0 = tuple(
        jnp.where(eye > 0, ed_ref[0, hh],
                  jnp.where(asym > 0, ed_ref[1, hh], ed_ref[255, hh]))
        for hh in range(_H))
    reach = jnp.minimum(asym + eye, 1.0).astype(jnp.bfloat16)

    def bfs_cond(st):
        t, _, _, changed = st
        return jnp.logical_and(changed, t < _N)

    def bfs_body(st):
        t, r, bs, _ = st
        cnt = jax.lax.dot_general(
            r, asym16, (((1,), (0,)), ((), ())),
            preferred_element_type=jnp.float32)
        rn = jnp.maximum(r, (cnt > 0.0).astype(jnp.bfloat16))
        new = rn > r
        tc = jnp.minimum(t, 255)
        bs = tuple(
            jnp.where(new, ed_ref[tc, hh], b) for hh, b in enumerate(bs))
        return t + 1, rn, bs, jnp.sum((rn - r).astype(jnp.float32)) > 0.0

    _, _, biases, _ = jax.lax.while_loop(
        bfs_cond, bfs_body,
        (jnp.int32(2), reach, bias0, jnp.bool_(True)))

    A = ahat
    xg = xf_ref[0]
    h = _dot(A, _dot(xg, c1w_ref[...])) + c1b_ref[...]
    h = _lrelu(h * _BNI * bn1g_ref[...] + bn1b_ref[...])

    isc = np.float32(1.0 / np.sqrt(_DH))
    for l in range(_NL):
        h = _dot(A, _dot(h, gw_ref[l])) + gb_ref[l]
        h = _lrelu(h * _BNI * bng_ref[l] + bnb_ref[l])
        x0 = h
        q = (_dot(x0, wq_ref[l]) + bq_ref[l]) * isc
        kk = _dot(x0, wk_ref[l]) + bk_ref[l]
        v = _dot(x0, wv_ref[l]) + bv_ref[l]
        outs = []
        for hh in range(_H):
            sl = slice(hh * _DH, (hh + 1) * _DH)
            sc = jax.lax.dot_general(
                q[:, sl], kk[:, sl], (((1,), (1,)), ((), ())),
                precision=jax.lax.Precision.DEFAULT,
                preferred_element_type=jnp.float32)
            sc = sc + biases[hh]
            m = jnp.max(sc, axis=-1, keepdims=True)
            e = jnp.exp(sc - m)
            attn = e / jnp.sum(e, axis=-1, keepdims=True)
            outs.append(jax.lax.dot_general(
                attn, v[:, sl], (((1,), (0,)), ((), ())),
                precision=jax.lax.Precision.DEFAULT,
                preferred_element_type=jnp.float32))
        o = jnp.concatenate(outs, axis=1)
        o = _dot(o, wo_ref[l]) + bo_ref[l]
        y = _ln(x0 + o, l1g_ref[l], l1b_ref[l])
        f = _dot(jnp.maximum(_dot(y, f1w_ref[l]) + f1b_ref[l], 0.0),
                 f2w_ref[l]) + f2b_ref[l]
        y = _ln(y + f, l2g_ref[l], l2b_ref[l])
        h = h + y

    xsum = jnp.sum(h, axis=0, keepdims=True)
    z = jnp.concatenate([xsum / _N, xsum], axis=1)
    z = _lrelu(_dot(z, q1w_ref[...]) * _BNI * q6g_ref[...] + q6b_ref[...])
    z = _lrelu((_dot(z, q2w_ref[...]) + q2b_ref[...]) * _BNI * q7g_ref[...]
               + q7b_ref[...])
    out_ref[0] = _dot(z, q3w_ref[...]) + q3b_ref[...]


def _full(shape):
    n = len(shape)
    return pl.BlockSpec(shape, lambda b: (0,) * n)


def _perb(shape):
    n = len(shape)
    return pl.BlockSpec((1,) + shape, lambda b: (b,) + (0,) * n)


def kernel(x, params):
    p = params
    inner = -2.0 * jnp.einsum('bcn,bcm->bnm', x, x)
    xx = jnp.sum(x ** 2, axis=1, keepdims=True)
    pd = -xx - inner - jnp.transpose(xx, (0, 2, 1))
    xf = jnp.transpose(x, (0, 2, 1))

    def st(fmt, reshape=None):
        a = jnp.stack([p[fmt % l] for l in range(_NL)])
        return a.reshape(reshape) if reshape else a

    args = [
        pd, xf, p['edge_dis'],
        p['conv1_W'], p['conv1_b'].reshape(1, _C),
        p['bn1_g'].reshape(1, _C), p['bn1_b'].reshape(1, _C),
        st('l%d_gcn_W'), st('l%d_gcn_b', (_NL, 1, _C)),
        st('l%d_bn_g', (_NL, 1, _C)), st('l%d_bn_b', (_NL, 1, _C)),
        st('l%d_Wq'), st('l%d_Wk'), st('l%d_Wv'), st('l%d_Wo'),
        st('l%d_bq', (_NL, 1, _C)), st('l%d_bk', (_NL, 1, _C)),
        st('l%d_bv', (_NL, 1, _C)), st('l%d_bo', (_NL, 1, _C)),
        st('l%d_ln1_g', (_NL, 1, _C)), st('l%d_ln1_b', (_NL, 1, _C)),
        st('l%d_fc1_W'), st('l%d_fc1_b', (_NL, 1, _FFN)),
        st('l%d_fc2_W'), st('l%d_fc2_b', (_NL, 1, _C)),
        st('l%d_ln2_g', (_NL, 1, _C)), st('l%d_ln2_b', (_NL, 1, _C)),
        p['lin1_W'],
        p['bn6_g'].reshape(1, 2 * _C), p['bn6_b'].reshape(1, 2 * _C),
        p['lin2_W'], p['lin2_b'].reshape(1, 2 * _C),
        p['bn7_g'].reshape(1, 2 * _C), p['bn7_b'].reshape(1, 2 * _C),
        p['lin3_W'], p['lin3_b'].reshape(1, 40),
    ]

    in_specs = [
        _perb((_N, _N)), _perb((_N, 3)),
        pl.BlockSpec(memory_space=pltpu.SMEM),
    ] + [_full(a.shape) for a in args[3:]]

    out = pl.pallas_call(
        _fwd_kernel,
        grid=(_B,),
        in_specs=in_specs,
        out_specs=_perb((1, 40)),
        out_shape=jax.ShapeDtypeStruct((_B, 1, 40), jnp.float32),
        compiler_params=pltpu.CompilerParams(
            dimension_semantics=("parallel",)),
    )(*args)
    return out.reshape(_B, 40)
